# baseline (speedup 1.0000x reference)
import functools

import jax
import jax.numpy as jnp
from jax.experimental import pallas as pl
from jax.experimental.pallas import tpu as pltpu

BG_IDX = 0
ADD_IDX = 23
RHO = 0.85
KMAX_SEM = 5
LAMBDA_VIS = 0.5
LAMBDA_SEM = 0.7
LAMBDA_OBS = 0.3
SCALE = 20.0
WINDOW = 5
ADD_BIAS = -1.5
L_ADD_BG = 2.5
L_ADD_LOWCONF = 1.0
L_ADD_ENT = 0.8
L_ADD_MISMATCH = 2.0
ADD_SCALE = 2.0
ADD_STEP_THRESH = 0.35
EPS = 1e-8


def _l2n(x):
    n = jnp.sqrt(jnp.sum(x * x, axis=-1, keepdims=True))
    return x / jnp.maximum(n, EPS)


def _rowdot_t(x, y):
    ones_row = jnp.ones((1, x.shape[1]), jnp.float32)
    return jax.lax.dot_general(ones_row, x * y, (((1,), (1,)), ((), ())),
                               preferred_element_type=jnp.float32)


def _rinv(s):
    return jax.lax.rsqrt(jnp.maximum(s, EPS * EPS))


def _fused_kernel(ff_ref, vs_ref, ss_ref, so_ref, unc_ref, sp_ref, ep_ref,
                  sm_ref, tp_ref, fep_ref, p_scr, spn_scr, epn_scr,
                  *, block_t, num_blocks):
    i = pl.program_id(0)

    @pl.when(i == 0)
    def _():
        spn_scr[...] = _l2n(sp_ref[...])
        epn_scr[...] = _l2n(ep_ref[...])

    ffb = ff_ref[...]
    vsb = vs_ref[...]
    ssb = ss_ref[...]
    sob = so_ref[...]
    unc = unc_ref[...]

    u_dim = unc.shape[-1]
    unc_norm = jnp.sqrt(_rowdot_t(unc, unc)) / (float(u_dim) ** 0.5)
    sem_conf = jnp.clip(jnp.exp(-unc_norm), 0.25, 1.0)

    c1 = _rinv(_rowdot_t(ffb, ffb))
    c2 = LAMBDA_VIS * _rinv(_rowdot_t(vsb, vsb))
    c3 = (LAMBDA_SEM * sem_conf) * _rinv(_rowdot_t(ssb, ssb))
    c4 = (LAMBDA_OBS * sem_conf) * _rinv(_rowdot_t(sob, sob))
    ct = jnp.concatenate([c1, c2, c3, c4], axis=0).T

    q = (ffb * ct[:, 0:1] + vsb * ct[:, 1:2]
         + ssb * ct[:, 2:3] + sob * ct[:, 3:4])

    inv_q = _rinv(_rowdot_t(q, q))

    simT = inv_q * jax.lax.dot_general(spn_scr[...], q, (((1,), (1,)), ((), ())),
                                       preferred_element_type=jnp.float32)

    simw = simT
    m_list = []
    for _ in range(KMAX_SEM):
        m = jnp.max(simw, axis=0, keepdims=True)
        m_list.append(m)
        simw = jnp.where(simw >= m, -3.0, simw)

    t = SCALE * simT
    zmax = SCALE * m_list[0]
    ez = jnp.exp(t - zmax)
    denom = jnp.sum(ez, axis=0, keepdims=True)
    alpha_max = 1.0 / denom

    cmass = jnp.zeros_like(denom)
    z_sum = jnp.zeros_like(denom)
    sc_acc = jnp.zeros_like(denom)
    for j in range(KMAX_SEM):
        v = jnp.exp(SCALE * m_list[j] - zmax) / denom
        cmass = cmass + v
        prev = cmass - v
        rv = jnp.where(prev < RHO, v, 0.0)
        z_sum = z_sum + rv
        sc_acc = sc_acc + rv * m_list[j]
    step_score = sc_acc / jnp.maximum(z_sum, EPS)

    tlT = (SCALE * inv_q) * jax.lax.dot_general(
        epn_scr[...], q, (((1,), (1,)), ((), ())),
        preferred_element_type=jnp.float32)
    tlT = tlT - jnp.max(tlT, axis=0, keepdims=True)
    etl = jnp.exp(tlT)
    type_prob = etl * (1.0 / jnp.sum(etl, axis=0, keepdims=True))
    p = jnp.maximum(type_prob, EPS)
    c_dim = type_prob.shape[0]
    ent = -jnp.sum(p * jnp.log(p), axis=0, keepdims=True) / jnp.log(float(max(c_dim, 2)))
    bg_prob = type_prob[:1, :]
    add_logit = (ADD_BIAS + L_ADD_BG * bg_prob + L_ADD_LOWCONF * (1.0 - alpha_max)
                 + L_ADD_ENT * ent
                 + L_ADD_MISMATCH * jnp.maximum(ADD_STEP_THRESH - step_score, 0.0))
    add_gate = jax.nn.sigmoid(ADD_SCALE * add_logit)
    sub_c = jax.lax.broadcasted_iota(jnp.int32, type_prob.shape, 0)
    onehot_add = jnp.where(sub_c == ADD_IDX, 1.0, 0.0)
    p_adj = type_prob * (1.0 - add_gate) + add_gate * onehot_add

    p_scr[i] = p_adj

    @pl.when(i == num_blocks - 1)
    def _():
        left = WINDOW // 2
        for b in range(num_blocks):
            pa_b = p_scr[b]
            if b > 0:
                lh = p_scr[b - 1][:, block_t - left:]
            else:
                lh = jnp.concatenate([pa_b[:, :1]] * left, axis=1)
            if b < num_blocks - 1:
                rh = p_scr[b + 1][:, :left]
            else:
                rh = jnp.concatenate([pa_b[:, block_t - 1:]] * left, axis=1)
            ext = jnp.concatenate([lh, pa_b, rh], axis=1)
            sm = ext[:, :block_t]
            for k in range(1, 2 * left + 1):
                sm = jnp.maximum(sm, ext[:, k:k + block_t])
            col = slice(b * block_t, (b + 1) * block_t)
            sm_ref[:, col] = sm
            mx = jnp.max(sm, axis=0, keepdims=True)
            c_iota = jax.lax.broadcasted_iota(jnp.int32, sm.shape, 0)
            tp = jnp.min(jnp.where(sm == mx, c_iota, sm.shape[0]),
                         axis=0, keepdims=True)
            tp_ref[:, col] = tp
            fep_ref[:, col] = (tp != BG_IDX).astype(jnp.float32)


@jax.jit
def kernel(frame_features, vis_short_seq, sem_short_seq, semantic_obs_seq,
           uncertainty_trace_seq, step_prototypes, error_prototypes):
    t_total, d = frame_features.shape
    s, _ = step_prototypes.shape
    c, _ = error_prototypes.shape
    u = uncertainty_trace_seq.shape[-1]
    block_t = 1024
    num_blocks = t_total // block_t

    feat_spec = pl.BlockSpec((block_t, d), lambda i: (i, 0))
    full = lambda shape: pl.BlockSpec(shape, lambda i: (0,) * len(shape))

    smoothed, tp, fep = pl.pallas_call(
        functools.partial(_fused_kernel, block_t=block_t, num_blocks=num_blocks),
        grid=(num_blocks,),
        in_specs=[feat_spec, feat_spec, feat_spec, feat_spec,
                  pl.BlockSpec((block_t, u), lambda i: (i, 0)),
                  full((s, d)), full((c, d))],
        out_specs=[full((c, t_total)), full((1, t_total)), full((1, t_total))],
        out_shape=[jax.ShapeDtypeStruct((c, t_total), jnp.float32),
                   jax.ShapeDtypeStruct((1, t_total), jnp.int32),
                   jax.ShapeDtypeStruct((1, t_total), jnp.float32)],
        scratch_shapes=[pltpu.VMEM((num_blocks, c, block_t), jnp.float32),
                        pltpu.VMEM((s, d), jnp.float32),
                        pltpu.VMEM((c, d), jnp.float32)],
    )(frame_features, vis_short_seq, sem_short_seq, semantic_obs_seq,
      uncertainty_trace_seq, step_prototypes, error_prototypes)
    return smoothed, tp[0], fep[0]


# speedup vs baseline: 1.0880x; 1.0880x over previous
import functools

import jax
import jax.numpy as jnp
from jax.experimental import pallas as pl
from jax.experimental.pallas import tpu as pltpu

BG_IDX = 0
ADD_IDX = 23
RHO = 0.85
KMAX_SEM = 5
LAMBDA_VIS = 0.5
LAMBDA_SEM = 0.7
LAMBDA_OBS = 0.3
SCALE = 20.0
WINDOW = 5
ADD_BIAS = -1.5
L_ADD_BG = 2.5
L_ADD_LOWCONF = 1.0
L_ADD_ENT = 0.8
L_ADD_MISMATCH = 2.0
ADD_SCALE = 2.0
ADD_STEP_THRESH = 0.35
EPS = 1e-8


def _l2n(x):
    n = jnp.sqrt(jnp.sum(x * x, axis=-1, keepdims=True))
    return x / jnp.maximum(n, EPS)


def _rowdot_t(x, y):
    ones_row = jnp.ones((1, x.shape[1]), jnp.float32)
    return jax.lax.dot_general(ones_row, x * y, (((1,), (1,)), ((), ())),
                               preferred_element_type=jnp.float32)


def _row_sumsq_mxu(x):
    ones = jnp.ones((x.shape[1], 128), jnp.float32)
    s = jax.lax.dot_general(x * x, ones, (((1,), (0,)), ((), ())),
                            preferred_element_type=jnp.float32)
    return s[:, :1]


def _inv_norm(x):
    return jax.lax.rsqrt(jnp.maximum(_row_sumsq_mxu(x), EPS * EPS))


def _rinv(s):
    return jax.lax.rsqrt(jnp.maximum(s, EPS * EPS))


def _fused_kernel(ff_ref, vs_ref, ss_ref, so_ref, unc_ref, sp_ref, ep_ref,
                  sm_ref, tp_ref, fep_ref, p_scr, spn_scr, epn_scr,
                  *, block_t, num_blocks):
    i = pl.program_id(0)

    @pl.when(i == 0)
    def _():
        spn_scr[...] = _l2n(sp_ref[...])
        epn_scr[...] = _l2n(ep_ref[...])

    ffb = ff_ref[...]
    vsb = vs_ref[...]
    ssb = ss_ref[...]
    sob = so_ref[...]
    unc = unc_ref[...]

    u_dim = unc.shape[-1]
    unc_norm = jnp.sqrt(_row_sumsq_mxu(unc)) / (float(u_dim) ** 0.5)
    sem_conf = jnp.clip(jnp.exp(-unc_norm), 0.25, 1.0)

    q = (ffb * _inv_norm(ffb)
         + vsb * (LAMBDA_VIS * _inv_norm(vsb))
         + ssb * ((LAMBDA_SEM * sem_conf) * _inv_norm(ssb))
         + sob * ((LAMBDA_OBS * sem_conf) * _inv_norm(sob)))

    inv_q = _rinv(_rowdot_t(q, q))

    simT = inv_q * jax.lax.dot_general(spn_scr[...], q, (((1,), (1,)), ((), ())),
                                       preferred_element_type=jnp.float32)

    simw = simT
    m_list = []
    for _ in range(KMAX_SEM):
        m = jnp.max(simw, axis=0, keepdims=True)
        m_list.append(m)
        simw = jnp.where(simw >= m, -3.0, simw)

    t = SCALE * simT
    zmax = SCALE * m_list[0]
    ez = jnp.exp(t - zmax)
    denom = jnp.sum(ez, axis=0, keepdims=True)
    alpha_max = 1.0 / denom

    cmass = jnp.zeros_like(denom)
    z_sum = jnp.zeros_like(denom)
    sc_acc = jnp.zeros_like(denom)
    for j in range(KMAX_SEM):
        v = jnp.exp(SCALE * m_list[j] - zmax) / denom
        cmass = cmass + v
        prev = cmass - v
        rv = jnp.where(prev < RHO, v, 0.0)
        z_sum = z_sum + rv
        sc_acc = sc_acc + rv * m_list[j]
    step_score = sc_acc / jnp.maximum(z_sum, EPS)

    tlT = (SCALE * inv_q) * jax.lax.dot_general(
        epn_scr[...], q, (((1,), (1,)), ((), ())),
        preferred_element_type=jnp.float32)
    tlT = tlT - jnp.max(tlT, axis=0, keepdims=True)
    etl = jnp.exp(tlT)
    type_prob = etl * (1.0 / jnp.sum(etl, axis=0, keepdims=True))
    p = jnp.maximum(type_prob, EPS)
    c_dim = type_prob.shape[0]
    ent = -jnp.sum(p * jnp.log(p), axis=0, keepdims=True) / jnp.log(float(max(c_dim, 2)))
    bg_prob = type_prob[:1, :]
    add_logit = (ADD_BIAS + L_ADD_BG * bg_prob + L_ADD_LOWCONF * (1.0 - alpha_max)
                 + L_ADD_ENT * ent
                 + L_ADD_MISMATCH * jnp.maximum(ADD_STEP_THRESH - step_score, 0.0))
    add_gate = jax.nn.sigmoid(ADD_SCALE * add_logit)
    sub_c = jax.lax.broadcasted_iota(jnp.int32, type_prob.shape, 0)
    onehot_add = jnp.where(sub_c == ADD_IDX, 1.0, 0.0)
    p_adj = type_prob * (1.0 - add_gate) + add_gate * onehot_add

    p_scr[i] = p_adj

    @pl.when(i == num_blocks - 1)
    def _():
        left = WINDOW // 2
        for b in range(num_blocks):
            pa_b = p_scr[b]
            if b > 0:
                lh = p_scr[b - 1][:, block_t - left:]
            else:
                lh = jnp.concatenate([pa_b[:, :1]] * left, axis=1)
            if b < num_blocks - 1:
                rh = p_scr[b + 1][:, :left]
            else:
                rh = jnp.concatenate([pa_b[:, block_t - 1:]] * left, axis=1)
            ext = jnp.concatenate([lh, pa_b, rh], axis=1)
            sm = ext[:, :block_t]
            for k in range(1, 2 * left + 1):
                sm = jnp.maximum(sm, ext[:, k:k + block_t])
            col = slice(b * block_t, (b + 1) * block_t)
            sm_ref[:, col] = sm
            mx = jnp.max(sm, axis=0, keepdims=True)
            c_iota = jax.lax.broadcasted_iota(jnp.int32, sm.shape, 0)
            tp = jnp.min(jnp.where(sm == mx, c_iota, sm.shape[0]),
                         axis=0, keepdims=True)
            tp_ref[:, col] = tp
            fep_ref[:, col] = (tp != BG_IDX).astype(jnp.float32)


@jax.jit
def kernel(frame_features, vis_short_seq, sem_short_seq, semantic_obs_seq,
           uncertainty_trace_seq, step_prototypes, error_prototypes):
    t_total, d = frame_features.shape
    s, _ = step_prototypes.shape
    c, _ = error_prototypes.shape
    u = uncertainty_trace_seq.shape[-1]
    block_t = 1024
    num_blocks = t_total // block_t

    feat_spec = pl.BlockSpec((block_t, d), lambda i: (i, 0))
    full = lambda shape: pl.BlockSpec(shape, lambda i: (0,) * len(shape))

    smoothed, tp, fep = pl.pallas_call(
        functools.partial(_fused_kernel, block_t=block_t, num_blocks=num_blocks),
        grid=(num_blocks,),
        in_specs=[feat_spec, feat_spec, feat_spec, feat_spec,
                  pl.BlockSpec((block_t, u), lambda i: (i, 0)),
                  full((s, d)), full((c, d))],
        out_specs=[full((c, t_total)), full((1, t_total)), full((1, t_total))],
        out_shape=[jax.ShapeDtypeStruct((c, t_total), jnp.float32),
                   jax.ShapeDtypeStruct((1, t_total), jnp.int32),
                   jax.ShapeDtypeStruct((1, t_total), jnp.float32)],
        scratch_shapes=[pltpu.VMEM((num_blocks, c, block_t), jnp.float32),
                        pltpu.VMEM((s, d), jnp.float32),
                        pltpu.VMEM((c, d), jnp.float32)],
    )(frame_features, vis_short_seq, sem_short_seq, semantic_obs_seq,
      uncertainty_trace_seq, step_prototypes, error_prototypes)
    return smoothed, tp[0], fep[0]
